# Initial kernel scaffold; baseline (speedup 1.0000x reference)
import jax
import jax.numpy as jnp
from jax.experimental import pallas as pl


def kernel(x, aa_table, pos_table, gamma, beta):
    raise NotImplementedError("write your pallas kernel here")


# speedup vs baseline: 3.5330x; 3.5330x over previous
import functools

import jax
import jax.numpy as jnp
from jax import lax
from jax.experimental import pallas as pl
from jax.experimental.pallas import tpu as pltpu
from jax.experimental.pallas import tpu_sc as plsc

BATCH = 16384
SEQ = 33
VOCAB = 27
DIM = 128
ROWS = VOCAB * SEQ
N = BATCH * SEQ
NC, NS = 2, 16
NW = NC * NS
BPW = N // NW
CHUNK = 128
NCHUNK = BPW // CHUNK


def _prep_body(x_ref, aa_ref, pos_ref, gamma_ref, beta_ref, table_ref, idx_ref):
    aa = aa_ref[...]
    pos = pos_ref[...]
    emb = aa[:, None, :] + pos[None, :, :]
    mean = jnp.mean(emb, axis=-1, keepdims=True)
    var = jnp.mean((emb - mean) ** 2, axis=-1, keepdims=True)
    normed = (emb - mean) * lax.rsqrt(var + 1e-5)
    table_ref[...] = normed * gamma_ref[...][None, None, :] + beta_ref[...][None, None, :]

    s = lax.broadcasted_iota(jnp.int32, (BATCH, SEQ), 1)
    idx_ref[...] = x_ref[...] * SEQ + s


@jax.jit
def _prep(x, aa_table, pos_table, gamma, beta):
    return pl.pallas_call(
        _prep_body,
        out_shape=(
            jax.ShapeDtypeStruct((VOCAB, SEQ, DIM), jnp.float32),
            jax.ShapeDtypeStruct((BATCH, SEQ), jnp.int32),
        ),
    )(x, aa_table, pos_table, gamma, beta)


def _gather_body(table_hbm, idx_hbm, out_hbm, idx_v, rows_v, sem):
    wid = lax.axis_index("s") * NC + lax.axis_index("c")
    base = wid * BPW
    pltpu.sync_copy(idx_hbm.at[pl.ds(base, BPW)], idx_v)

    def body(g, carry):
        cbase = g * CHUNK
        pltpu.async_copy(
            table_hbm.at[idx_v.at[pl.ds(cbase, CHUNK)]], rows_v, sem
        ).wait()
        pltpu.sync_copy(rows_v, out_hbm.at[pl.ds(base + cbase, CHUNK)])
        return carry

    lax.fori_loop(0, NCHUNK, body, 0)


_gather = pl.kernel(
    _gather_body,
    out_type=jax.ShapeDtypeStruct((N, DIM), jnp.float32),
    mesh=plsc.VectorSubcoreMesh(core_axis_name="c", subcore_axis_name="s"),
    scratch_types=[
        pltpu.VMEM((BPW,), jnp.int32),
        pltpu.VMEM((CHUNK, DIM), jnp.float32),
        pltpu.SemaphoreType.DMA,
    ],
)


def kernel(x, aa_table, pos_table, gamma, beta):
    table, idx2d = _prep(x, aa_table, pos_table, gamma, beta)
    out = _gather(table.reshape(ROWS, DIM), idx2d.reshape(N))
    return out.reshape(BATCH, SEQ, DIM)


# speedup vs baseline: 3.7181x; 1.0524x over previous
import functools

import jax
import jax.numpy as jnp
from jax import lax
from jax.experimental import pallas as pl
from jax.experimental.pallas import tpu as pltpu
from jax.experimental.pallas import tpu_sc as plsc

BATCH = 16384
SEQ = 33
VOCAB = 27
DIM = 128
ROWS = VOCAB * SEQ
N = BATCH * SEQ
NC, NS = 2, 16
NW = NC * NS
BPW = N // NW
CHUNK = 128
NCHUNK = BPW // CHUNK


def _prep_body(x_ref, aa_ref, pos_ref, gamma_ref, beta_ref, table_ref, idx_ref):
    aa = aa_ref[...]
    pos = pos_ref[...]
    emb = aa[:, None, :] + pos[None, :, :]
    mean = jnp.mean(emb, axis=-1, keepdims=True)
    var = jnp.mean((emb - mean) ** 2, axis=-1, keepdims=True)
    normed = (emb - mean) * lax.rsqrt(var + 1e-5)
    table_ref[...] = normed * gamma_ref[...][None, None, :] + beta_ref[...][None, None, :]

    s = lax.broadcasted_iota(jnp.int32, (BATCH, SEQ), 1)
    idx_ref[...] = x_ref[...] * SEQ + s


@jax.jit
def _prep(x, aa_table, pos_table, gamma, beta):
    return pl.pallas_call(
        _prep_body,
        out_shape=(
            jax.ShapeDtypeStruct((VOCAB, SEQ, DIM), jnp.float32),
            jax.ShapeDtypeStruct((BATCH, SEQ), jnp.int32),
        ),
    )(x, aa_table, pos_table, gamma, beta)


NBUF = 4
NGROUP = NCHUNK // NBUF


def _gather_body(table_hbm, idx_hbm, out_hbm, idx_v, rows_v, *sems):
    sem_g, sem_s = sems[:NBUF], sems[NBUF:]
    wid = lax.axis_index("s") * NC + lax.axis_index("c")
    base = wid * BPW
    pltpu.sync_copy(idx_hbm.at[pl.ds(base, BPW)], idx_v)

    def g_copy(b, g):
        return pltpu.make_async_copy(
            table_hbm.at[idx_v.at[pl.ds(g * CHUNK, CHUNK)]],
            rows_v.at[b], sem_g[b])

    def s_copy(b, g):
        return pltpu.make_async_copy(
            rows_v.at[b], out_hbm.at[pl.ds(base + g * CHUNK, CHUNK)],
            sem_s[b])

    def slot(b, g, first, last):
        g_copy(b, g).wait()
        s_copy(b, g).start()
        pb = (b - 1) % NBUF
        if not first:
            s_copy(pb, g - 1).wait()
        if not last:
            g_copy(pb, g + NBUF - 1).start()

    for h in range(NBUF - 1):
        g_copy(h, h).start()

    for b in range(NBUF):
        slot(b, b, first=(b == 0), last=False)

    def body(gi, carry):
        for b in range(NBUF):
            slot(b, gi * NBUF + b, first=False, last=False)
        return carry

    lax.fori_loop(1, NGROUP - 1, body, 0)

    for b in range(NBUF):
        g = (NGROUP - 1) * NBUF + b
        slot(b, g, first=False, last=(b >= 1))
    s_copy(NBUF - 1, NCHUNK - 1).wait()


_gather = pl.kernel(
    _gather_body,
    out_type=jax.ShapeDtypeStruct((N, DIM), jnp.float32),
    mesh=plsc.VectorSubcoreMesh(core_axis_name="c", subcore_axis_name="s"),
    scratch_types=[
        pltpu.VMEM((BPW,), jnp.int32),
        pltpu.VMEM((NBUF, CHUNK, DIM), jnp.float32),
    ] + [pltpu.SemaphoreType.DMA] * (2 * NBUF),
)


def kernel(x, aa_table, pos_table, gamma, beta):
    table, idx2d = _prep(x, aa_table, pos_table, gamma, beta)
    out = _gather(table.reshape(ROWS, DIM), idx2d.reshape(N))
    return out.reshape(BATCH, SEQ, DIM)
